# Initial kernel scaffold; baseline (speedup 1.0000x reference)
import jax
import jax.numpy as jnp
from jax.experimental import pallas as pl


def kernel(x, edge_index, W1, b1, W2, b2):
    raise NotImplementedError("write your pallas kernel here")


# speedup vs baseline: 5.7490x; 5.7490x over previous
import functools

import jax
import jax.numpy as jnp
from jax import lax
from jax.experimental import pallas as pl
from jax.experimental.pallas import tpu as pltpu
from jax.experimental.pallas import tpu_sc as plsc

NC = 2
NS = 16
EW = 128


def _sc_mesh():
    return plsc.VectorSubcoreMesh(core_axis_name="c", subcore_axis_name="s")


def _make_hist_kernel(NPAD, K):
    rps = NPAD // NS

    @functools.partial(
        pl.kernel,
        mesh=_sc_mesh(),
        out_type=(
            jax.ShapeDtypeStruct((NC, NPAD), jnp.float32),
            jax.ShapeDtypeStruct((NC, NPAD), jnp.float32),
        ),
        scratch_types=[
            pltpu.VMEM((K, EW), jnp.int32),
            pltpu.VMEM((K, EW), jnp.int32),
            pltpu.VMEM((EW,), jnp.float32),
            pltpu.VMEM_SHARED((NPAD,), jnp.float32),
            pltpu.VMEM_SHARED((NPAD,), jnp.float32),
        ],
    )
    def hist(src_hbm, dst_hbm, z1_hbm, do_out, di_out,
             src_v, dst_v, ones_v, do_sh, di_sh):
        cid = lax.axis_index("c")
        sid = lax.axis_index("s")
        wid = sid * NC + cid
        pltpu.sync_copy(z1_hbm, do_sh.at[pl.ds(sid * rps, rps)])
        pltpu.sync_copy(z1_hbm, di_sh.at[pl.ds(sid * rps, rps)])
        for i in range(EW // 16):
            ones_v[pl.ds(i * 16, 16)] = jnp.ones((16,), jnp.float32)
        pltpu.sync_copy(src_hbm.at[wid], src_v)
        pltpu.sync_copy(dst_hbm.at[wid], dst_v)
        plsc.subcore_barrier()

        def body(j, carry):
            pltpu.sync_copy(ones_v, do_sh.at[src_v.at[j]], add=True)
            pltpu.sync_copy(ones_v, di_sh.at[dst_v.at[j]], add=True)
            return carry

        lax.fori_loop(0, K, body, 0)
        plsc.subcore_barrier()
        pltpu.sync_copy(do_sh.at[pl.ds(sid * rps, rps)],
                        do_out.at[cid, pl.ds(sid * rps, rps)])
        pltpu.sync_copy(di_sh.at[pl.ds(sid * rps, rps)],
                        di_out.at[cid, pl.ds(sid * rps, rps)])

    return hist


def _make_edge_kernel(NPAD, K, Dh):
    rps = NPAD // NS

    @functools.partial(
        pl.kernel,
        mesh=_sc_mesh(),
        out_type=(
            jax.ShapeDtypeStruct((NC, NPAD, Dh), jnp.float32),
            jax.ShapeDtypeStruct((NC, NPAD), jnp.float32),
        ),
        scratch_types=[
            pltpu.VMEM((K, EW), jnp.int32),
            pltpu.VMEM((K, EW), jnp.int32),
            pltpu.VMEM((EW, Dh), jnp.float32),
            pltpu.VMEM((EW,), jnp.float32),
            pltpu.VMEM_SHARED((NPAD, Dh), jnp.float32),
            pltpu.VMEM_SHARED((NPAD,), jnp.float32),
            pltpu.SemaphoreType.DMA,
            pltpu.SemaphoreType.DMA,
        ],
    )
    def edge(src_hbm, dst_hbm, y_hbm, nd_hbm, z2_hbm, z1_hbm,
             agg_out, c_out, src_v, dst_v, rows_v, ndv_v,
             agg_sh, c_sh, sem_y, sem_n):
        cid = lax.axis_index("c")
        sid = lax.axis_index("s")
        wid = sid * NC + cid
        pltpu.sync_copy(src_hbm.at[wid], src_v)
        pltpu.sync_copy(dst_hbm.at[wid], dst_v)
        pltpu.sync_copy(z2_hbm, agg_sh.at[pl.ds(sid * rps, rps)])
        pltpu.sync_copy(z1_hbm, c_sh.at[pl.ds(sid * rps, rps)])
        plsc.subcore_barrier()

        def body(j, carry):
            pltpu.async_copy(y_hbm.at[src_v.at[j]], rows_v, sem_y).wait()
            pltpu.sync_copy(rows_v, agg_sh.at[dst_v.at[j]], add=True)
            pltpu.async_copy(nd_hbm.at[dst_v.at[j]], ndv_v, sem_n).wait()
            pltpu.sync_copy(ndv_v, c_sh.at[src_v.at[j]], add=True)
            return carry

        lax.fori_loop(0, K, body, 0)
        plsc.subcore_barrier()
        pltpu.sync_copy(agg_sh.at[pl.ds(sid * rps, rps)],
                        agg_out.at[cid, pl.ds(sid * rps, rps)])
        pltpu.sync_copy(c_sh.at[pl.ds(sid * rps, rps)],
                        c_out.at[cid, pl.ds(sid * rps, rps)])

    return edge


def _dense1(do_p, di_p, xpad, W1, NPAD, BN):
    Din = xpad.shape[1]
    Dh = W1.shape[1]

    def body(do_ref, di_ref, x_ref, w1_ref, y_ref, ns_ref, nd_ref):
        deg_o = do_ref[0, :] + do_ref[1, :]
        deg_i = di_ref[0, :] + di_ref[1, :]
        ns = jnp.where(deg_o > 0, lax.rsqrt(jnp.maximum(deg_o, 1e-12)), 0.0)
        nd = jnp.where(deg_i > 0, lax.rsqrt(jnp.maximum(deg_i, 1e-12)), 0.0)
        ns_ref[0, :] = ns
        nd_ref[0, :] = nd
        y_ref[...] = jnp.dot(x_ref[...], w1_ref[...],
                             preferred_element_type=jnp.float32) * ns[:, None]

    grid = (NPAD // BN,)
    return pl.pallas_call(
        body,
        grid=grid,
        in_specs=[
            pl.BlockSpec((NC, BN), lambda i: (0, i)),
            pl.BlockSpec((NC, BN), lambda i: (0, i)),
            pl.BlockSpec((BN, Din), lambda i: (i, 0)),
            pl.BlockSpec((Din, Dh), lambda i: (0, 0)),
        ],
        out_specs=[
            pl.BlockSpec((BN, Dh), lambda i: (i, 0)),
            pl.BlockSpec((1, BN), lambda i: (0, i)),
            pl.BlockSpec((1, BN), lambda i: (0, i)),
        ],
        out_shape=[
            jax.ShapeDtypeStruct((NPAD, Dh), jnp.float32),
            jax.ShapeDtypeStruct((1, NPAD), jnp.float32),
            jax.ShapeDtypeStruct((1, NPAD), jnp.float32),
        ],
    )(do_p, di_p, xpad, W1)


def _dense2(agg_p, c_p, ns, nd, b1, W2, b2, NPAD, N, BN):
    Dh = agg_p.shape[2]
    ncls = W2.shape[1]
    grid_n = NPAD // BN

    def body(agg_ref, c_ref, ns_ref, nd_ref, b1_ref, w2_ref, b2_ref,
             out_ref, s_ref):
        i = pl.program_id(0)
        agg = agg_ref[0] + agg_ref[1]
        h1 = jnp.maximum(nd_ref[0, :][:, None] * agg + b1_ref[0, :][None, :],
                         0.0)
        w = ns_ref[0, :] * (c_ref[0, :] + c_ref[1, :])
        row = i * BN + lax.broadcasted_iota(jnp.int32, (1, BN), 1)[0]
        w = jnp.where(row < N, w, 0.0)
        part = jnp.dot(w[None, :], h1, preferred_element_type=jnp.float32)

        @pl.when(i == 0)
        def _():
            s_ref[...] = part

        @pl.when(i > 0)
        def _():
            s_ref[...] = s_ref[...] + part

        @pl.when(i == grid_n - 1)
        def _():
            out_ref[...] = jnp.dot(s_ref[...] * (1.0 / N), w2_ref[...],
                                   preferred_element_type=jnp.float32) \
                + b2_ref[...]

    return pl.pallas_call(
        body,
        grid=(grid_n,),
        in_specs=[
            pl.BlockSpec((NC, BN, Dh), lambda i: (0, i, 0)),
            pl.BlockSpec((NC, BN), lambda i: (0, i)),
            pl.BlockSpec((1, BN), lambda i: (0, i)),
            pl.BlockSpec((1, BN), lambda i: (0, i)),
            pl.BlockSpec((1, Dh), lambda i: (0, 0)),
            pl.BlockSpec((Dh, ncls), lambda i: (0, 0)),
            pl.BlockSpec((1, ncls), lambda i: (0, 0)),
        ],
        out_specs=pl.BlockSpec((1, ncls), lambda i: (0, 0)),
        out_shape=jax.ShapeDtypeStruct((1, ncls), jnp.float32),
        scratch_shapes=[pltpu.VMEM((1, Dh), jnp.float32)],
    )(agg_p, c_p, ns, nd, b1, W2, b2)


def kernel(x, edge_index, W1, b1, W2, b2):
    N, Din = x.shape
    Dh = W1.shape[1]
    E = edge_index.shape[1]
    NW = NC * NS
    NPAD = -(-N // 2048) * 2048
    K = -(-E // (NW * EW))
    if K % 2:
        K += 1
    EPAD = NW * K * EW

    src = edge_index[0]
    dst = edge_index[1]
    padv = jnp.full((EPAD - E,), N, jnp.int32)
    srcp = jnp.concatenate([src, padv]).reshape(NW, K, EW)
    dstp = jnp.concatenate([dst, padv]).reshape(NW, K, EW)
    xpad = jnp.pad(x, ((0, NPAD - N), (0, 0)))
    rps = NPAD // NS
    z2 = jnp.zeros((rps, Dh), jnp.float32)
    z1 = jnp.zeros((rps,), jnp.float32)

    do_p, di_p = _make_hist_kernel(NPAD, K)(srcp, dstp, z1)
    y, ns, nd = _dense1(do_p, di_p, xpad, W1, NPAD, 1024)
    agg_p, c_p = _make_edge_kernel(NPAD, K, Dh)(
        srcp, dstp, y, nd.reshape(NPAD), z2, z1)
    out = _dense2(agg_p, c_p, ns, nd, b1.reshape(1, Dh), W2,
                  b2.reshape(1, W2.shape[1]), NPAD, N, 1024)
    return out


# speedup vs baseline: 7.7248x; 1.3437x over previous
import functools

import jax
import jax.numpy as jnp
from jax import lax
from jax.experimental import pallas as pl
from jax.experimental.pallas import tpu as pltpu
from jax.experimental.pallas import tpu_sc as plsc

NC = 2
NS = 16
EW = 128


def _sc_mesh():
    return plsc.VectorSubcoreMesh(core_axis_name="c", subcore_axis_name="s")


def _make_hist_kernel(NPAD, K):
    rps = NPAD // NS

    @functools.partial(
        pl.kernel,
        mesh=_sc_mesh(),
        out_type=(
            jax.ShapeDtypeStruct((NC, NPAD), jnp.float32),
            jax.ShapeDtypeStruct((NC, NPAD), jnp.float32),
        ),
        scratch_types=[
            pltpu.VMEM((K, EW), jnp.int32),
            pltpu.VMEM((K, EW), jnp.int32),
            pltpu.VMEM((K, EW), jnp.float32),
            pltpu.VMEM_SHARED((NPAD,), jnp.float32),
            pltpu.VMEM_SHARED((NPAD,), jnp.float32),
        ],
    )
    def hist(src_hbm, dst_hbm, ones_hbm, z1_hbm, do_out, di_out,
             src_v, dst_v, ones_v, do_sh, di_sh):
        cid = lax.axis_index("c")
        sid = lax.axis_index("s")
        wid = sid * NC + cid
        pltpu.sync_copy(z1_hbm, do_sh.at[pl.ds(sid * rps, rps)])
        pltpu.sync_copy(z1_hbm, di_sh.at[pl.ds(sid * rps, rps)])
        pltpu.sync_copy(ones_hbm, ones_v)
        pltpu.sync_copy(src_hbm.at[wid], src_v)
        pltpu.sync_copy(dst_hbm.at[wid], dst_v)
        plsc.subcore_barrier()

        def body(j, carry):
            pltpu.sync_copy(ones_v.at[j], do_sh.at[src_v.at[j]], add=True)
            pltpu.sync_copy(ones_v.at[j], di_sh.at[dst_v.at[j]], add=True)
            return carry

        lax.fori_loop(0, K, body, 0)
        plsc.subcore_barrier()
        pltpu.sync_copy(do_sh.at[pl.ds(sid * rps, rps)],
                        do_out.at[cid, pl.ds(sid * rps, rps)])
        pltpu.sync_copy(di_sh.at[pl.ds(sid * rps, rps)],
                        di_out.at[cid, pl.ds(sid * rps, rps)])

    return hist


def _make_edge_kernel(NPAD, K, Dh):
    rps = NPAD // NS
    NB = 2
    P = 2
    K2 = K // P

    @functools.partial(
        pl.kernel,
        mesh=_sc_mesh(),
        out_type=(
            jax.ShapeDtypeStruct((NC, NPAD, Dh), jnp.float32),
            jax.ShapeDtypeStruct((NC, NPAD), jnp.float32),
        ),
        scratch_types=[
            pltpu.VMEM((K2, EW), jnp.int32),
            pltpu.VMEM((K2, EW), jnp.int32),
            pltpu.VMEM((NB, EW, Dh), jnp.float32),
            pltpu.VMEM((NB, EW), jnp.float32),
            pltpu.VMEM_SHARED((NPAD, Dh), jnp.float32),
            pltpu.VMEM_SHARED((NPAD,), jnp.float32),
            pltpu.SemaphoreType.DMA,
            pltpu.SemaphoreType.DMA,
            pltpu.SemaphoreType.DMA,
            pltpu.SemaphoreType.DMA,
        ],
    )
    def edge(src_hbm, dst_hbm, y_hbm, nd_hbm, z2_hbm, z1_hbm,
             agg_out, c_out, src_v, dst_v, rows_v, ndv_v,
             agg_sh, c_sh, sem_y0, sem_y1, sem_n0, sem_n1):
        cid = lax.axis_index("c")
        sid = lax.axis_index("s")
        wid = sid * NC + cid
        pltpu.sync_copy(z2_hbm, agg_sh.at[pl.ds(sid * rps, rps)])
        pltpu.sync_copy(z1_hbm, c_sh.at[pl.ds(sid * rps, rps)])
        plsc.subcore_barrier()
        semys = (sem_y0, sem_y1)
        semns = (sem_n0, sem_n1)

        def fire_y(j, b):
            pltpu.async_copy(y_hbm.at[src_v.at[j]], rows_v.at[b], semys[b])

        def drain_y(j, b):
            pltpu.make_async_copy(y_hbm.at[src_v.at[j]], rows_v.at[b],
                                  semys[b]).wait()

        def fire_n(j, b):
            pltpu.async_copy(nd_hbm.at[dst_v.at[j]], ndv_v.at[b], semns[b])

        def drain_n(j, b):
            pltpu.make_async_copy(nd_hbm.at[dst_v.at[j]], ndv_v.at[b],
                                  semns[b]).wait()

        for p in range(P):
            pltpu.sync_copy(src_hbm.at[wid, pl.ds(p * K2, K2)], src_v)
            pltpu.sync_copy(dst_hbm.at[wid, pl.ds(p * K2, K2)], dst_v)
            for b in range(NB):
                fire_y(b, b)
                fire_n(b, b)

            def body(tt, carry):
                for b in range(NB):
                    j = tt * NB + b
                    drain_y(j, b)
                    pltpu.sync_copy(rows_v.at[b],
                                    agg_sh.at[dst_v.at[j]], add=True)
                    drain_n(j, b)
                    pltpu.sync_copy(ndv_v.at[b],
                                    c_sh.at[src_v.at[j]], add=True)

                    @pl.when(j + NB < K2)
                    def _():
                        fire_y(j + NB, b)
                        fire_n(j + NB, b)
                return carry

            lax.fori_loop(0, K2 // NB, body, 0)
        plsc.subcore_barrier()
        pltpu.sync_copy(agg_sh.at[pl.ds(sid * rps, rps)],
                        agg_out.at[cid, pl.ds(sid * rps, rps)])
        pltpu.sync_copy(c_sh.at[pl.ds(sid * rps, rps)],
                        c_out.at[cid, pl.ds(sid * rps, rps)])

    return edge


def _dense1(do_p, di_p, xpad, W1, NPAD, BN):
    Din = xpad.shape[1]
    Dh = W1.shape[1]

    def body(do_ref, di_ref, x_ref, w1_ref, y_ref, ns_ref, nd_ref):
        deg_o = do_ref[0, :] + do_ref[1, :]
        deg_i = di_ref[0, :] + di_ref[1, :]
        ns = jnp.where(deg_o > 0, lax.rsqrt(jnp.maximum(deg_o, 1e-12)), 0.0)
        nd = jnp.where(deg_i > 0, lax.rsqrt(jnp.maximum(deg_i, 1e-12)), 0.0)
        ns_ref[0, :] = ns
        nd_ref[0, :] = nd
        y_ref[...] = jnp.dot(x_ref[...], w1_ref[...],
                             preferred_element_type=jnp.float32) * ns[:, None]

    grid = (NPAD // BN,)
    return pl.pallas_call(
        body,
        grid=grid,
        in_specs=[
            pl.BlockSpec((NC, BN), lambda i: (0, i)),
            pl.BlockSpec((NC, BN), lambda i: (0, i)),
            pl.BlockSpec((BN, Din), lambda i: (i, 0)),
            pl.BlockSpec((Din, Dh), lambda i: (0, 0)),
        ],
        out_specs=[
            pl.BlockSpec((BN, Dh), lambda i: (i, 0)),
            pl.BlockSpec((1, BN), lambda i: (0, i)),
            pl.BlockSpec((1, BN), lambda i: (0, i)),
        ],
        out_shape=[
            jax.ShapeDtypeStruct((NPAD, Dh), jnp.float32),
            jax.ShapeDtypeStruct((1, NPAD), jnp.float32),
            jax.ShapeDtypeStruct((1, NPAD), jnp.float32),
        ],
    )(do_p, di_p, xpad, W1)


def _dense2(agg_p, c_p, ns, nd, b1, W2, b2, NPAD, N, BN):
    Dh = agg_p.shape[2]
    ncls = W2.shape[1]
    grid_n = NPAD // BN

    def body(agg_ref, c_ref, ns_ref, nd_ref, b1_ref, w2_ref, b2_ref,
             out_ref, s_ref):
        i = pl.program_id(0)
        agg = agg_ref[0] + agg_ref[1]
        h1 = jnp.maximum(nd_ref[0, :][:, None] * agg + b1_ref[0, :][None, :],
                         0.0)
        w = ns_ref[0, :] * (c_ref[0, :] + c_ref[1, :])
        row = i * BN + lax.broadcasted_iota(jnp.int32, (1, BN), 1)[0]
        w = jnp.where(row < N, w, 0.0)
        part = jnp.dot(w[None, :], h1, preferred_element_type=jnp.float32)

        @pl.when(i == 0)
        def _():
            s_ref[...] = part

        @pl.when(i > 0)
        def _():
            s_ref[...] = s_ref[...] + part

        @pl.when(i == grid_n - 1)
        def _():
            out_ref[...] = jnp.dot(s_ref[...] * (1.0 / N), w2_ref[...],
                                   preferred_element_type=jnp.float32) \
                + b2_ref[...]

    return pl.pallas_call(
        body,
        grid=(grid_n,),
        in_specs=[
            pl.BlockSpec((NC, BN, Dh), lambda i: (0, i, 0)),
            pl.BlockSpec((NC, BN), lambda i: (0, i)),
            pl.BlockSpec((1, BN), lambda i: (0, i)),
            pl.BlockSpec((1, BN), lambda i: (0, i)),
            pl.BlockSpec((1, Dh), lambda i: (0, 0)),
            pl.BlockSpec((Dh, ncls), lambda i: (0, 0)),
            pl.BlockSpec((1, ncls), lambda i: (0, 0)),
        ],
        out_specs=pl.BlockSpec((1, ncls), lambda i: (0, 0)),
        out_shape=jax.ShapeDtypeStruct((1, ncls), jnp.float32),
        scratch_shapes=[pltpu.VMEM((1, Dh), jnp.float32)],
    )(agg_p, c_p, ns, nd, b1, W2, b2)


def kernel(x, edge_index, W1, b1, W2, b2):
    N, Din = x.shape
    Dh = W1.shape[1]
    E = edge_index.shape[1]
    NW = NC * NS
    NPAD = -(-N // 2048) * 2048
    K = -(-E // (NW * EW))
    K = -(-K // 4) * 4
    EPAD = NW * K * EW

    src = edge_index[0]
    dst = edge_index[1]
    padv = jnp.full((EPAD - E,), N, jnp.int32)
    srcp = jnp.concatenate([src, padv]).reshape(NW, K, EW)
    dstp = jnp.concatenate([dst, padv]).reshape(NW, K, EW)
    xpad = jnp.pad(x, ((0, NPAD - N), (0, 0)))
    rps = NPAD // NS
    z2 = jnp.zeros((rps, Dh), jnp.float32)
    z1 = jnp.zeros((rps,), jnp.float32)

    ones = jnp.ones((K, EW), jnp.float32)
    do_p, di_p = _make_hist_kernel(NPAD, K)(srcp, dstp, ones, z1)
    y, ns, nd = _dense1(do_p, di_p, xpad, W1, NPAD, 1024)
    agg_p, c_p = _make_edge_kernel(NPAD, K, Dh)(
        srcp, dstp, y, nd.reshape(NPAD), z2, z1)
    out = _dense2(agg_p, c_p, ns, nd, b1.reshape(1, Dh), W2,
                  b2.reshape(1, W2.shape[1]), NPAD, N, 1024)
    return out
